# baseline (speedup 1.0000x reference)
import functools

import jax
import jax.numpy as jnp
from jax import lax
from jax.experimental import pallas as pl
from jax.experimental.pallas import tpu as pltpu

N = 17
D = 768
BS = 4096
B = 8
BN = B * N


def _pfd_block(m_ref, m1_ref, out_ref):
    m2 = m_ref[...]
    pwf2 = m2 * m1_ref[...]

    dots = lax.dot_general(
        m2, pwf2, (((1,), (1,)), ((), ())), preferred_element_type=jnp.float32
    )

    na_col = jnp.sqrt(jnp.sum(m2 * m2, axis=1, keepdims=True))
    sq = pwf2 * pwf2
    nb_row = jnp.sqrt(
        lax.dot_general(
            jnp.ones((1, D), jnp.float32), sq, (((1,), (1,)), ((), ())),
            preferred_element_type=jnp.float32,
        )
    )
    denom = jnp.maximum(na_col * nb_row, 1e-8)
    sim = dots / denom

    r_blk = lax.broadcasted_iota(jnp.int32, (BN, BN), 0) // N
    c_idx = lax.broadcasted_iota(jnp.int32, (BN, BN), 1)
    valid = r_blk == (c_idx // N)
    simm = jnp.where(valid, sim, -jnp.inf)

    amax = jnp.argmax(simm, axis=1).astype(jnp.int32)
    oh = (amax[:, None] == c_idx).astype(jnp.float32)

    gathered = lax.dot_general(
        oh, pwf2, (((1,), (0,)), ((), ())), preferred_element_type=jnp.float32
    )
    out_ref[...] = m2 + gathered


@jax.jit
def kernel(matrix, matrix1):
    m2 = matrix.reshape(BS * N, D)
    m12 = matrix1.reshape(BS * N, D)
    out = pl.pallas_call(
        _pfd_block,
        grid=(BS // B,),
        in_specs=[
            pl.BlockSpec((BN, D), lambda i: (i, 0)),
            pl.BlockSpec((BN, D), lambda i: (i, 0)),
        ],
        out_specs=pl.BlockSpec((BN, D), lambda i: (i, 0)),
        out_shape=jax.ShapeDtypeStruct((BS * N, D), jnp.float32),
        compiler_params=pltpu.CompilerParams(
            dimension_semantics=("parallel",),
        ),
    )(m2, m12)
    return out.reshape(BS, N, D)


# speedup vs baseline: 1.1896x; 1.1896x over previous
import jax
import jax.numpy as jnp
from jax import lax
from jax.experimental import pallas as pl
from jax.experimental.pallas import tpu as pltpu

N = 17
D = 768
BS = 4096
B = 8


def _pfd_block(m_ref, m1_ref, out_ref):
    m = m_ref[...]
    pwf = m * m1_ref[...]

    nb = jnp.sqrt(jnp.sum(pwf * pwf, axis=2, keepdims=True))
    pwfn = pwf / jnp.maximum(nb, 1e-8)

    dots = lax.dot_general(
        m, pwfn, (((2,), (2,)), ((0,), (0,))),
        preferred_element_type=jnp.float32,
        precision=lax.Precision.HIGHEST,
    )

    ind = jnp.argmax(dots, axis=2).astype(jnp.int32)
    c_idx = lax.broadcasted_iota(jnp.int32, (B, N, N), 2)
    oh = (ind[:, :, None] == c_idx).astype(jnp.float32)

    gathered = lax.dot_general(
        oh, pwf, (((2,), (1,)), ((0,), (0,))),
        preferred_element_type=jnp.float32,
    )
    out_ref[...] = m + gathered


@jax.jit
def kernel(matrix, matrix1):
    return pl.pallas_call(
        _pfd_block,
        grid=(BS // B,),
        in_specs=[
            pl.BlockSpec((B, N, D), lambda i: (i, 0, 0)),
            pl.BlockSpec((B, N, D), lambda i: (i, 0, 0)),
        ],
        out_specs=pl.BlockSpec((B, N, D), lambda i: (i, 0, 0)),
        out_shape=jax.ShapeDtypeStruct((BS, N, D), jnp.float32),
        compiler_params=pltpu.CompilerParams(
            dimension_semantics=("parallel",),
        ),
    )(matrix, matrix1)
